# baseline (speedup 1.0000x reference)
import functools

import jax
import jax.numpy as jnp
from jax import lax
from jax.experimental import pallas as pl
from jax.experimental.pallas import tpu as pltpu
from jax.experimental.pallas import tpu_sc as plsc

TOPK = 9
LANES = 16
NWORKERS = 32


def _sc_decode(H, W, N, nq, nt, dz, xc2, par, xtf, zc2):
    per_w = nq // NWORKERS
    groups = per_w // LANES
    mesh = plsc.VectorSubcoreMesh(core_axis_name="c", subcore_axis_name="s")

    @functools.partial(
        pl.kernel,
        mesh=mesh,
        out_type=jax.ShapeDtypeStruct((nq, dz), jnp.float32),
        compiler_params=pltpu.CompilerParams(needs_layout_passes=False),
        scratch_types=[
            pltpu.VMEM((2 * N,), jnp.float32),
            pltpu.VMEM((LANES,), jnp.float32),
            pltpu.VMEM((2 * per_w,), jnp.float32),
            pltpu.VMEM((TOPK, LANES), jnp.float32),
            pltpu.VMEM((TOPK, LANES), jnp.float32),
            pltpu.VMEM((TOPK, LANES), jnp.int32),
            pltpu.VMEM((TOPK, LANES), jnp.int32),
            pltpu.VMEM((TOPK, LANES, dz), jnp.float32),
            pltpu.VMEM((TOPK, LANES, dz), jnp.float32),
            pltpu.VMEM((LANES, dz), jnp.float32),
            pltpu.VMEM((LANES, dz), jnp.float32),
            pltpu.SemaphoreType.DMA,
            pltpu.SemaphoreType.DMA,
            pltpu.SemaphoreType.DMA,
            pltpu.SemaphoreType.DMA,
        ],
    )
    def body(xc_h, par_h, xt_h, zc_h, out_h,
             xcbuf, par_r, qbuf, wbuf0, wbuf1, ibuf0, ibuf1,
             rows0, rows1, obuf0, obuf1, sem0, sem1, semo0, semo1):
        wbufs, ibufs = (wbuf0, wbuf1), (ibuf0, ibuf1)
        rowss, obufs = (rows0, rows1), (obuf0, obuf1)
        cid = lax.axis_index("c")
        sid = lax.axis_index("s")
        wid = sid * 2 + cid
        base = wid * per_w
        lane = lax.iota(jnp.int32, LANES)

        staging = [
            pltpu.async_copy(xc_h.at[pl.ds(0, 2 * N)], xcbuf, sem0),
            pltpu.async_copy(par_h, par_r, sem0),
            pltpu.async_copy(xt_h.at[pl.ds(2 * base, 2 * per_w)], qbuf, sem0),
        ]
        for c in staging:
            c.wait()
        nhil = par_r[...]
        zeros16 = jnp.zeros((LANES,), jnp.int32)
        ones16 = jnp.full((LANES,), 1, jnp.int32)

        def select_and_fire(g, buf, sem):
            qx = plsc.load_gather(qbuf, [g * 2 * LANES + 2 * lane])
            qy = plsc.load_gather(qbuf, [g * 2 * LANES + 2 * lane + 1])
            gid = base + g * LANES + lane
            row_base = lax.div(gid, nt) * N
            fx = (qx + 1.0) * ((H - 1) * 0.5)
            fy = (qy + 1.0) * ((W - 1) * 0.5)
            sx = jnp.clip((fx + 0.5).astype(jnp.int32) - 2, 0, H - 5)
            sy = jnp.clip((fy + 0.5).astype(jnp.int32) - 2, 0, W - 5)

            dx2 = []
            colb = []
            for i in range(5):
                gxi = plsc.load_gather(xcbuf, [2 * ((sx + i) * W)])
                d = qx - gxi
                dx2.append(d * d)
                colb.append(row_base + (sx + i) * W + sy)
            dy2 = []
            for j in range(5):
                gyj = plsc.load_gather(xcbuf, [2 * (sy + j) + 1])
                d = qy - gyj
                dy2.append(d * d)

            NC = 25
            K = []
            Ci = []
            for i in range(5):
                for j in range(5):
                    K.append(dx2[i] + dy2[j])
                    Ci.append(colb[i] + j)

            R = [jnp.full((LANES,), NC - 1 - c, jnp.int32) for c in range(NC)]
            for a in range(NC):
                for b in range(a + 1, NC):
                    t = (K[a] <= K[b]).astype(jnp.int32)
                    R[b] = R[b] + t
                    R[a] = R[a] - t

            wbuf = wbufs[buf]
            ibuf = ibufs[buf]
            for c in range(NC):
                m = R[c] < TOPK
                slot = jnp.where(m, R[c], 0)
                plsc.store_scatter(ibuf, [slot, lane], Ci[c], mask=m)
                plsc.store_scatter(wbuf, [slot, lane],
                                   jnp.exp(K[c] * nhil), mask=m)
            return [pltpu.async_copy(zc_h.at[ibuf.at[s]], rowss[buf].at[s], sem)
                    for s in range(TOPK)]

        def weighted_sum(g, buf, semo):
            wbuf = wbufs[buf]
            rows = rowss[buf]
            obuf = obufs[buf]

            def qbody(q, qcarry):
                qi = jnp.broadcast_to(q, (LANES,))
                acc = [None] * (dz // LANES)
                for s in range(TOPK):
                    si = jnp.full((LANES,), s, jnp.int32)
                    wq = plsc.load_gather(wbuf, [si, qi])
                    for c in range(dz // LANES):
                        t = wq * rows[s, q, pl.ds(c * LANES, LANES)]
                        acc[c] = t if s == 0 else acc[c] + t
                for c in range(dz // LANES):
                    obuf[q, pl.ds(c * LANES, LANES)] = acc[c]
                return qcarry

            lax.fori_loop(0, LANES, qbody, 0)
            return pltpu.async_copy(
                obuf, out_h.at[pl.ds(base + g * LANES, LANES)], semo)

        def pair(h, carry):
            g0 = h * 2
            g1 = g0 + 1
            gath0 = select_and_fire(g0, 0, sem0)
            gath1 = select_and_fire(g1, 1, sem1)
            for c in gath0:
                c.wait()
            out0 = weighted_sum(g0, 0, semo0)
            for c in gath1:
                c.wait()
            out1 = weighted_sum(g1, 1, semo1)
            out0.wait()
            out1.wait()
            return carry

        lax.fori_loop(0, groups // 2, pair, 0)

    return body(xc2, par, xtf, zc2)


def kernel(xc, zc, xt, lengthscale_param):
    b, H, W, dx = xc.shape
    dz = zc.shape[-1]
    nt = xt.shape[1]
    N = H * W
    nq = b * nt
    lengthscale = 1e-05 + jax.nn.softplus(lengthscale_param)
    nhil = -0.5 / (lengthscale[0] * lengthscale[0])
    par = jnp.broadcast_to(nhil, (LANES,)).astype(jnp.float32)
    xc2 = xc.reshape(b * N * dx)
    xtf = xt.reshape(nq * dx)
    zc2 = zc.reshape(b * N, dz)
    out = _sc_decode(H, W, N, nq, nt, dz, xc2, par, xtf, zc2)
    return out.reshape(b, nt, dz)


# speedup vs baseline: 1.1801x; 1.1801x over previous
import functools

import jax
import jax.numpy as jnp
from jax import lax
from jax.experimental import pallas as pl
from jax.experimental.pallas import tpu as pltpu
from jax.experimental.pallas import tpu_sc as plsc

TOPK = 9
LANES = 16
NWORKERS = 32


def _sc_decode(H, W, N, nq, nt, dz, gxa, gya, par, qxa, qya, zc2):
    per_w = nq // NWORKERS
    groups = per_w // LANES
    mesh = plsc.VectorSubcoreMesh(core_axis_name="c", subcore_axis_name="s")

    @functools.partial(
        pl.kernel,
        mesh=mesh,
        out_type=jax.ShapeDtypeStruct((nq, dz), jnp.float32),
        compiler_params=pltpu.CompilerParams(needs_layout_passes=False),
        scratch_types=[
            pltpu.VMEM((H,), jnp.float32),
            pltpu.VMEM((W,), jnp.float32),
            pltpu.VMEM((LANES,), jnp.float32),
            pltpu.VMEM((per_w,), jnp.float32),
            pltpu.VMEM((per_w,), jnp.float32),
            pltpu.VMEM((TOPK, LANES), jnp.float32),
            pltpu.VMEM((TOPK, LANES), jnp.float32),
            pltpu.VMEM((TOPK, LANES), jnp.int32),
            pltpu.VMEM((TOPK, LANES), jnp.int32),
            pltpu.VMEM((TOPK, LANES, dz), jnp.float32),
            pltpu.VMEM((TOPK, LANES, dz), jnp.float32),
            pltpu.VMEM((LANES, dz), jnp.float32),
            pltpu.VMEM((LANES, dz), jnp.float32),
            pltpu.SemaphoreType.DMA,
            pltpu.SemaphoreType.DMA,
            pltpu.SemaphoreType.DMA,
            pltpu.SemaphoreType.DMA,
        ],
    )
    def body(gx_h, gy_h, par_h, qx_h, qy_h, zc_h, out_h,
             gx_r, gy_r, par_r, qx_r, qy_r, wbuf0, wbuf1, ibuf0, ibuf1,
             rows0, rows1, obuf0, obuf1, sem0, sem1, semo0, semo1):
        wbufs, ibufs = (wbuf0, wbuf1), (ibuf0, ibuf1)
        rowss, obufs = (rows0, rows1), (obuf0, obuf1)
        cid = lax.axis_index("c")
        sid = lax.axis_index("s")
        wid = sid * 2 + cid
        base = wid * per_w
        staging = [
            pltpu.async_copy(gx_h, gx_r, sem0),
            pltpu.async_copy(gy_h, gy_r, sem0),
            pltpu.async_copy(par_h, par_r, sem0),
            pltpu.async_copy(qx_h.at[pl.ds(base, per_w)], qx_r, sem0),
            pltpu.async_copy(qy_h.at[pl.ds(base, per_w)], qy_r, sem0),
        ]
        for c in staging:
            c.wait()
        nhil = par_r[...]
        lane = lax.iota(jnp.int32, LANES)

        def select_and_fire(g, buf, sem):
            qx = qx_r[pl.ds(g * LANES, LANES)]
            qy = qy_r[pl.ds(g * LANES, LANES)]
            gid = base + g * LANES + lane
            row_base = lax.div(gid, nt) * N
            fx = (qx + 1.0) * ((H - 1) * 0.5)
            fy = (qy + 1.0) * ((W - 1) * 0.5)
            sx = jnp.clip((fx + 0.5).astype(jnp.int32) - 2, 0, H - 5)
            sy = jnp.clip((fy + 0.5).astype(jnp.int32) - 2, 0, W - 5)

            dx2 = []
            colb = []
            for i in range(5):
                gxi = plsc.load_gather(gx_r, [sx + i])
                d = qx - gxi
                dx2.append(d * d)
                colb.append(row_base + (sx + i) * W + sy)
            dy2 = []
            for j in range(5):
                gyj = plsc.load_gather(gy_r, [sy + j])
                d = qy - gyj
                dy2.append(d * d)

            NC = 25
            K = []
            Ci = []
            for i in range(5):
                for j in range(5):
                    K.append(dx2[i] + dy2[j])
                    Ci.append(colb[i] + j)

            R = [jnp.full((LANES,), NC - 1 - c, jnp.int32) for c in range(NC)]
            for a in range(NC):
                for b in range(a + 1, NC):
                    t = (K[a] <= K[b]).astype(jnp.int32)
                    R[b] = R[b] + t
                    R[a] = R[a] - t

            wbuf = wbufs[buf]
            ibuf = ibufs[buf]
            for c in range(NC):
                m = R[c] < TOPK
                slot = jnp.where(m, R[c], 0)
                plsc.store_scatter(ibuf, [slot, lane], Ci[c], mask=m)
                plsc.store_scatter(wbuf, [slot, lane],
                                   jnp.exp(K[c] * nhil), mask=m)
            return [pltpu.async_copy(zc_h.at[ibuf.at[s]], rowss[buf].at[s], sem)
                    for s in range(TOPK)]

        def weighted_sum(g, buf, semo):
            wbuf = wbufs[buf]
            rows = rowss[buf]
            obuf = obufs[buf]

            @plsc.parallel_loop(0, LANES, unroll=2)
            def qbody(q):
                qi = jnp.broadcast_to(q, (LANES,))
                acc = [None] * (dz // LANES)
                for s in range(TOPK):
                    si = jnp.full((LANES,), s, jnp.int32)
                    wq = plsc.load_gather(wbuf, [si, qi])
                    for c in range(dz // LANES):
                        t = wq * rows[s, q, pl.ds(c * LANES, LANES)]
                        acc[c] = t if s == 0 else acc[c] + t
                for c in range(dz // LANES):
                    obuf[q, pl.ds(c * LANES, LANES)] = acc[c]

            return pltpu.async_copy(
                obuf, out_h.at[pl.ds(base + g * LANES, LANES)], semo)

        def pair(h, carry):
            g0 = h * 2
            g1 = g0 + 1
            gath0 = select_and_fire(g0, 0, sem0)
            gath1 = select_and_fire(g1, 1, sem1)
            for c in gath0:
                c.wait()
            out0 = weighted_sum(g0, 0, semo0)
            for c in gath1:
                c.wait()
            out1 = weighted_sum(g1, 1, semo1)
            out0.wait()
            out1.wait()
            return carry

        lax.fori_loop(0, groups // 2, pair, 0)

    return body(gxa, gya, par, qxa, qya, zc2)


def kernel(xc, zc, xt, lengthscale_param):
    b, H, W, dx = xc.shape
    dz = zc.shape[-1]
    nt = xt.shape[1]
    N = H * W
    nq = b * nt
    lengthscale = 1e-05 + jax.nn.softplus(lengthscale_param)
    nhil = -0.5 / (lengthscale[0] * lengthscale[0])
    par = jnp.broadcast_to(nhil, (LANES,)).astype(jnp.float32)
    gxa = xc[0, :, 0, 0]
    gya = xc[0, 0, :, 1]
    qxa = xt[:, :, 0].reshape(-1)
    qya = xt[:, :, 1].reshape(-1)
    zc2 = zc.reshape(b * N, dz)
    out = _sc_decode(H, W, N, nq, nt, dz, gxa, gya, par, qxa, qya, zc2)
    return out.reshape(b, nt, dz)


# speedup vs baseline: 1.4121x; 1.1967x over previous
import functools

import jax
import jax.numpy as jnp
from jax import lax
from jax.experimental import pallas as pl
from jax.experimental.pallas import tpu as pltpu
from jax.experimental.pallas import tpu_sc as plsc

TOPK = 9
LANES = 16
NWORKERS = 32


def _sc_decode(H, W, N, nq, nt, dz, gxa, gya, par, qxa, qya, zc2):
    per_w = nq // NWORKERS
    groups = per_w // LANES
    mesh = plsc.VectorSubcoreMesh(core_axis_name="c", subcore_axis_name="s")

    @functools.partial(
        pl.kernel,
        mesh=mesh,
        out_type=jax.ShapeDtypeStruct((nq, dz), jnp.float32),
        compiler_params=pltpu.CompilerParams(needs_layout_passes=False),
        scratch_types=[
            pltpu.VMEM((H,), jnp.float32),
            pltpu.VMEM((W,), jnp.float32),
            pltpu.VMEM((LANES,), jnp.float32),
            pltpu.VMEM((per_w,), jnp.float32),
            pltpu.VMEM((per_w,), jnp.float32),
            pltpu.VMEM((TOPK, LANES), jnp.float32),
            pltpu.VMEM((TOPK, LANES), jnp.float32),
            pltpu.VMEM((TOPK, LANES), jnp.int32),
            pltpu.VMEM((TOPK, LANES), jnp.int32),
            pltpu.VMEM((TOPK, LANES, dz), jnp.float32),
            pltpu.VMEM((TOPK, LANES, dz), jnp.float32),
            pltpu.VMEM((LANES, dz), jnp.float32),
            pltpu.VMEM((LANES, dz), jnp.float32),
            pltpu.SemaphoreType.DMA,
            pltpu.SemaphoreType.DMA,
            pltpu.SemaphoreType.DMA,
            pltpu.SemaphoreType.DMA,
        ],
    )
    def body(gx_h, gy_h, par_h, qx_h, qy_h, zc_h, out_h,
             gx_r, gy_r, par_r, qx_r, qy_r, wbuf0, wbuf1, ibuf0, ibuf1,
             rows0, rows1, obuf0, obuf1, sem0, sem1, semo0, semo1):
        wbufs, ibufs = (wbuf0, wbuf1), (ibuf0, ibuf1)
        rowss, obufs = (rows0, rows1), (obuf0, obuf1)
        cid = lax.axis_index("c")
        sid = lax.axis_index("s")
        wid = sid * 2 + cid
        base = wid * per_w
        staging = [
            pltpu.async_copy(gx_h, gx_r, sem0),
            pltpu.async_copy(gy_h, gy_r, sem0),
            pltpu.async_copy(par_h, par_r, sem0),
            pltpu.async_copy(qx_h.at[pl.ds(base, per_w)], qx_r, sem0),
            pltpu.async_copy(qy_h.at[pl.ds(base, per_w)], qy_r, sem0),
        ]
        for c in staging:
            c.wait()
        nhil = par_r[...]
        lane = lax.iota(jnp.int32, LANES)

        def select(g, buf):
            qx = qx_r[pl.ds(g * LANES, LANES)]
            qy = qy_r[pl.ds(g * LANES, LANES)]
            gid = base + g * LANES + lane
            row_base = lax.div(gid, nt) * N
            fx = (qx + 1.0) * ((H - 1) * 0.5)
            fy = (qy + 1.0) * ((W - 1) * 0.5)
            sx = jnp.clip((fx + 0.5).astype(jnp.int32) - 2, 0, H - 5)
            sy = jnp.clip((fy + 0.5).astype(jnp.int32) - 2, 0, W - 5)

            dx2 = []
            colb = []
            for i in range(5):
                gxi = plsc.load_gather(gx_r, [sx + i])
                d = qx - gxi
                dx2.append(d * d)
                colb.append(row_base + (sx + i) * W + sy)
            dy2 = []
            for j in range(5):
                gyj = plsc.load_gather(gy_r, [sy + j])
                d = qy - gyj
                dy2.append(d * d)

            NC = 25
            K = []
            Ci = []
            for i in range(5):
                for j in range(5):
                    K.append(dx2[i] + dy2[j])
                    Ci.append(colb[i] + j)

            R = [jnp.full((LANES,), NC - 1 - c, jnp.int32) for c in range(NC)]
            for a in range(NC):
                for b in range(a + 1, NC):
                    t = (K[a] <= K[b]).astype(jnp.int32)
                    R[b] = R[b] + t
                    R[a] = R[a] - t

            wbuf = wbufs[buf]
            ibuf = ibufs[buf]
            for c in range(NC):
                m = R[c] < TOPK
                slot = jnp.where(m, R[c], 0)
                plsc.store_scatter(ibuf, [slot, lane], Ci[c], mask=m)
                plsc.store_scatter(wbuf, [slot, lane],
                                   jnp.exp(K[c] * nhil), mask=m)

        def fire(buf, sem):
            for s in range(TOPK):
                pltpu.async_copy(zc_h.at[ibufs[buf].at[s]],
                                 rowss[buf].at[s], sem)

        def drain(buf, sem):
            for s in range(TOPK):
                pltpu.make_async_copy(zc_h.at[ibufs[buf].at[s]],
                                      rowss[buf].at[s], sem).wait()

        def weighted_sum(g, buf, semo):
            wbuf = wbufs[buf]
            rows = rowss[buf]
            obuf = obufs[buf]

            def qbody(qh, qcarry):
                for u in range(2):
                    q = qh * 2 + u
                    qi = jnp.broadcast_to(q, (LANES,))
                    acc = [None] * (dz // LANES)
                    for s in range(TOPK):
                        si = jnp.full((LANES,), s, jnp.int32)
                        wq = plsc.load_gather(wbuf, [si, qi])
                        for c in range(dz // LANES):
                            t = wq * rows[s, q, pl.ds(c * LANES, LANES)]
                            acc[c] = t if s == 0 else acc[c] + t
                    for c in range(dz // LANES):
                        obuf[q, pl.ds(c * LANES, LANES)] = acc[c]
                return qcarry

            lax.fori_loop(0, LANES // 2, qbody, 0)
            return pltpu.async_copy(
                obuf, out_h.at[pl.ds(base + g * LANES, LANES)], semo)

        select(0, 0)
        fire(0, sem0)
        select(1, 1)
        fire(1, sem1)

        def pair(k, carry):
            g0 = k * 2
            g1 = g0 + 1
            drain(0, sem0)
            out0 = weighted_sum(g0, 0, semo0)
            select(g0 + 2, 0)
            fire(0, sem0)
            drain(1, sem1)
            out1 = weighted_sum(g1, 1, semo1)
            select(g1 + 2, 1)
            fire(1, sem1)
            out0.wait()
            out1.wait()
            return carry

        lax.fori_loop(0, groups // 2 - 1, pair, 0)
        drain(0, sem0)
        out0 = weighted_sum(groups - 2, 0, semo0)
        drain(1, sem1)
        out1 = weighted_sum(groups - 1, 1, semo1)
        out0.wait()
        out1.wait()

    return body(gxa, gya, par, qxa, qya, zc2)


def kernel(xc, zc, xt, lengthscale_param):
    b, H, W, dx = xc.shape
    dz = zc.shape[-1]
    nt = xt.shape[1]
    N = H * W
    nq = b * nt
    lengthscale = 1e-05 + jax.nn.softplus(lengthscale_param)
    nhil = -0.5 / (lengthscale[0] * lengthscale[0])
    par = jnp.broadcast_to(nhil, (LANES,)).astype(jnp.float32)
    gxa = xc[0, :, 0, 0]
    gya = xc[0, 0, :, 1]
    qxa = xt[:, :, 0].reshape(-1)
    qya = xt[:, :, 1].reshape(-1)
    zc2 = zc.reshape(b * N, dz)
    out = _sc_decode(H, W, N, nq, nt, dz, gxa, gya, par, qxa, qya, zc2)
    return out.reshape(b, nt, dz)


# speedup vs baseline: 1.4573x; 1.0320x over previous
import functools

import jax
import jax.numpy as jnp
from jax import lax
from jax.experimental import pallas as pl
from jax.experimental.pallas import tpu as pltpu
from jax.experimental.pallas import tpu_sc as plsc

TOPK = 9
LANES = 16
NWORKERS = 32


def _sc_decode(H, W, N, nq, nt, dz, pack, xtt, zc2):
    per_w = nq // NWORKERS
    groups = per_w // LANES
    mesh = plsc.VectorSubcoreMesh(core_axis_name="c", subcore_axis_name="s")

    @functools.partial(
        pl.kernel,
        mesh=mesh,
        out_type=jax.ShapeDtypeStruct((nq, dz), jnp.float32),
        compiler_params=pltpu.CompilerParams(needs_layout_passes=False),
        scratch_types=[
            pltpu.VMEM((H + W + LANES,), jnp.float32),
            pltpu.VMEM((per_w,), jnp.float32),
            pltpu.VMEM((per_w,), jnp.float32),
            pltpu.VMEM((TOPK, LANES), jnp.float32),
            pltpu.VMEM((TOPK, LANES), jnp.float32),
            pltpu.VMEM((TOPK, LANES), jnp.int32),
            pltpu.VMEM((TOPK, LANES), jnp.int32),
            pltpu.VMEM((TOPK, LANES, dz), jnp.float32),
            pltpu.VMEM((TOPK, LANES, dz), jnp.float32),
            pltpu.VMEM((LANES, dz), jnp.float32),
            pltpu.VMEM((LANES, dz), jnp.float32),
            pltpu.SemaphoreType.DMA,
            pltpu.SemaphoreType.DMA,
            pltpu.SemaphoreType.DMA,
            pltpu.SemaphoreType.DMA,
        ],
    )
    def body(pack_h, xt_h, zc_h, out_h,
             pack_r, qx_r, qy_r, wbuf0, wbuf1, ibuf0, ibuf1,
             rows0, rows1, obuf0, obuf1, sem0, sem1, semo0, semo1):
        wbufs, ibufs = (wbuf0, wbuf1), (ibuf0, ibuf1)
        rowss, obufs = (rows0, rows1), (obuf0, obuf1)
        cid = lax.axis_index("c")
        sid = lax.axis_index("s")
        wid = sid * 2 + cid
        base = wid * per_w
        staging = [
            pltpu.async_copy(pack_h, pack_r, sem0),
            pltpu.async_copy(xt_h.at[0, pl.ds(base, per_w)], qx_r, sem0),
            pltpu.async_copy(xt_h.at[1, pl.ds(base, per_w)], qy_r, sem0),
        ]
        for c in staging:
            c.wait()
        nhil = pack_r[pl.ds(H + W, LANES)]
        lane = lax.iota(jnp.int32, LANES)

        def select(g, buf):
            qx = qx_r[pl.ds(g * LANES, LANES)]
            qy = qy_r[pl.ds(g * LANES, LANES)]
            gid = base + g * LANES + lane
            row_base = lax.div(gid, nt) * N
            fx = (qx + 1.0) * ((H - 1) * 0.5)
            fy = (qy + 1.0) * ((W - 1) * 0.5)
            sx = jnp.clip((fx + 0.5).astype(jnp.int32) - 2, 0, H - 5)
            sy = jnp.clip((fy + 0.5).astype(jnp.int32) - 2, 0, W - 5)

            dx2 = []
            colb = []
            for i in range(5):
                gxi = plsc.load_gather(pack_r, [sx + i])
                d = qx - gxi
                dx2.append(d * d)
                colb.append(row_base + (sx + i) * W + sy)
            dy2 = []
            for j in range(5):
                gyj = plsc.load_gather(pack_r, [H + sy + j])
                d = qy - gyj
                dy2.append(d * d)

            NC = 25
            K = []
            Ci = []
            for i in range(5):
                for j in range(5):
                    K.append(dx2[i] + dy2[j])
                    Ci.append(colb[i] + j)

            R = [jnp.full((LANES,), NC - 1 - c, jnp.int32) for c in range(NC)]
            for a in range(NC):
                for b in range(a + 1, NC):
                    t = (K[a] <= K[b]).astype(jnp.int32)
                    R[b] = R[b] + t
                    R[a] = R[a] - t

            wbuf = wbufs[buf]
            ibuf = ibufs[buf]
            for c in range(NC):
                m = R[c] < TOPK
                slot = jnp.where(m, R[c], 0)
                plsc.store_scatter(ibuf, [slot, lane], Ci[c], mask=m)
                plsc.store_scatter(wbuf, [slot, lane],
                                   jnp.exp(K[c] * nhil), mask=m)

        def fire(buf, sem):
            for s in range(TOPK):
                pltpu.async_copy(zc_h.at[ibufs[buf].at[s]],
                                 rowss[buf].at[s], sem)

        def drain(buf, sem):
            for s in range(TOPK):
                pltpu.make_async_copy(zc_h.at[ibufs[buf].at[s]],
                                      rowss[buf].at[s], sem).wait()

        def weighted_sum(g, buf, semo):
            wbuf = wbufs[buf]
            rows = rowss[buf]
            obuf = obufs[buf]

            def qbody(q, qcarry):
                qi = jnp.broadcast_to(q, (LANES,))
                acc = [None] * (dz // LANES)
                for s in range(TOPK):
                    si = jnp.full((LANES,), s, jnp.int32)
                    wq = plsc.load_gather(wbuf, [si, qi])
                    for c in range(dz // LANES):
                        t = wq * rows[s, q, pl.ds(c * LANES, LANES)]
                        acc[c] = t if s == 0 else acc[c] + t
                for c in range(dz // LANES):
                    obuf[q, pl.ds(c * LANES, LANES)] = acc[c]
                return qcarry

            lax.fori_loop(0, LANES, qbody, 0)
            return pltpu.async_copy(
                obuf, out_h.at[pl.ds(base + g * LANES, LANES)], semo)

        select(0, 0)
        fire(0, sem0)
        select(1, 1)
        fire(1, sem1)

        def pair(k, carry):
            g0 = k * 2
            g1 = g0 + 1
            drain(0, sem0)
            out0 = weighted_sum(g0, 0, semo0)
            select(g0 + 2, 0)
            fire(0, sem0)
            drain(1, sem1)
            out1 = weighted_sum(g1, 1, semo1)
            select(g1 + 2, 1)
            fire(1, sem1)
            out0.wait()
            out1.wait()
            return carry

        lax.fori_loop(0, groups // 2 - 1, pair, 0)
        drain(0, sem0)
        out0 = weighted_sum(groups - 2, 0, semo0)
        drain(1, sem1)
        out1 = weighted_sum(groups - 1, 1, semo1)
        out0.wait()
        out1.wait()

    return body(pack, xtt, zc2)


def kernel(xc, zc, xt, lengthscale_param):
    b, H, W, dx = xc.shape
    dz = zc.shape[-1]
    nt = xt.shape[1]
    N = H * W
    nq = b * nt
    lengthscale = 1e-05 + jax.nn.softplus(lengthscale_param)
    nhil = -0.5 / (lengthscale[0] * lengthscale[0])
    pack = jnp.concatenate([
        xc[0, :, 0, 0],
        xc[0, 0, :, 1],
        jnp.broadcast_to(nhil, (LANES,)).astype(jnp.float32),
    ])
    xtt = xt.reshape(nq, dx).T
    zc2 = zc.reshape(b * N, dz)
    out = _sc_decode(H, W, N, nq, nt, dz, pack, xtt, zc2)
    return out.reshape(b, nt, dz)


# speedup vs baseline: 1.7764x; 1.2190x over previous
import functools

import jax
import jax.numpy as jnp
from jax import lax
from jax.experimental import pallas as pl
from jax.experimental.pallas import tpu as pltpu
from jax.experimental.pallas import tpu_sc as plsc

TOPK = 9
LANES = 16
NWORKERS = 32


def _sc_decode(H, W, N, nq, nt, dz, pack, xtt, zc2):
    per_w = nq // NWORKERS
    groups = per_w // LANES
    mesh = plsc.VectorSubcoreMesh(core_axis_name="c", subcore_axis_name="s")

    @functools.partial(
        pl.kernel,
        mesh=mesh,
        out_type=jax.ShapeDtypeStruct((nq, dz), jnp.float32),
        compiler_params=pltpu.CompilerParams(needs_layout_passes=False),
        scratch_types=[
            pltpu.VMEM((H + W + LANES,), jnp.float32),
            pltpu.VMEM((per_w,), jnp.float32),
            pltpu.VMEM((per_w,), jnp.float32),
            pltpu.VMEM((TOPK, LANES), jnp.float32),
            pltpu.VMEM((TOPK, LANES), jnp.float32),
            pltpu.VMEM((TOPK, LANES), jnp.int32),
            pltpu.VMEM((TOPK, LANES), jnp.int32),
            pltpu.VMEM((TOPK, LANES, dz), jnp.float32),
            pltpu.VMEM((TOPK, LANES, dz), jnp.float32),
            pltpu.VMEM((LANES, dz), jnp.float32),
            pltpu.VMEM((LANES, dz), jnp.float32),
            pltpu.SemaphoreType.DMA,
            pltpu.SemaphoreType.DMA,
            pltpu.SemaphoreType.DMA,
            pltpu.SemaphoreType.DMA,
        ],
    )
    def body(pack_h, xt_h, zc_h, out_h,
             pack_r, qx_r, qy_r, wbuf0, wbuf1, ibuf0, ibuf1,
             rows0, rows1, obuf0, obuf1, sem0, sem1, semo0, semo1):
        wbufs, ibufs = (wbuf0, wbuf1), (ibuf0, ibuf1)
        rowss, obufs = (rows0, rows1), (obuf0, obuf1)
        cid = lax.axis_index("c")
        sid = lax.axis_index("s")
        wid = sid * 2 + cid
        base = wid * per_w
        staging = [
            pltpu.async_copy(pack_h, pack_r, sem0),
            pltpu.async_copy(xt_h.at[0, pl.ds(base, per_w)], qx_r, sem0),
            pltpu.async_copy(xt_h.at[1, pl.ds(base, per_w)], qy_r, sem0),
        ]
        for c in staging:
            c.wait()
        nhil = pack_r[pl.ds(H + W, LANES)]
        lane = lax.iota(jnp.int32, LANES)

        def select(g, buf):
            qx = qx_r[pl.ds(g * LANES, LANES)]
            qy = qy_r[pl.ds(g * LANES, LANES)]
            gid = base + g * LANES + lane
            row_base = lax.div(gid, nt) * N
            fx = (qx + 1.0) * ((H - 1) * 0.5)
            fy = (qy + 1.0) * ((W - 1) * 0.5)
            sx = jnp.clip((fx + 0.5).astype(jnp.int32) - 2, 0, H - 5)
            sy = jnp.clip((fy + 0.5).astype(jnp.int32) - 2, 0, W - 5)

            dx2 = []
            colb = []
            for i in range(5):
                gxi = plsc.load_gather(pack_r, [sx + i])
                d = qx - gxi
                dx2.append(d * d)
                colb.append(row_base + (sx + i) * W + sy)
            dy2 = []
            for j in range(5):
                gyj = plsc.load_gather(pack_r, [H + sy + j])
                d = qy - gyj
                dy2.append(d * d)

            NC = 25
            K = []
            Ci = []
            for i in range(5):
                for j in range(5):
                    K.append(dx2[i] + dy2[j])
                    Ci.append(colb[i] + j)

            BK = [jnp.full((LANES,), 1e30, jnp.float32) for _ in range(TOPK)]
            BI = [jnp.zeros((LANES,), jnp.int32) for _ in range(TOPK)]
            for c in range(NC):
                ck, ci = K[c], Ci[c]
                for s in range(TOPK):
                    lt = ck < BK[s]
                    BK[s], ck = jnp.where(lt, ck, BK[s]), jnp.where(lt, BK[s], ck)
                    BI[s], ci = jnp.where(lt, ci, BI[s]), jnp.where(lt, BI[s], ci)

            wbuf = wbufs[buf]
            ibuf = ibufs[buf]
            for s in range(TOPK):
                ibuf[s, :] = BI[s]
                wbuf[s, :] = jnp.exp(BK[s] * nhil)

        def fire(buf, sem):
            for s in range(TOPK):
                pltpu.async_copy(zc_h.at[ibufs[buf].at[s]],
                                 rowss[buf].at[s], sem)

        def drain(buf, sem):
            for s in range(TOPK):
                pltpu.make_async_copy(zc_h.at[ibufs[buf].at[s]],
                                      rowss[buf].at[s], sem).wait()

        def weighted_sum(g, buf, semo):
            wbuf = wbufs[buf]
            rows = rowss[buf]
            obuf = obufs[buf]

            def qbody(q, qcarry):
                qi = jnp.broadcast_to(q, (LANES,))
                acc = [None] * (dz // LANES)
                for s in range(TOPK):
                    si = jnp.full((LANES,), s, jnp.int32)
                    wq = plsc.load_gather(wbuf, [si, qi])
                    for c in range(dz // LANES):
                        t = wq * rows[s, q, pl.ds(c * LANES, LANES)]
                        acc[c] = t if s == 0 else acc[c] + t
                for c in range(dz // LANES):
                    obuf[q, pl.ds(c * LANES, LANES)] = acc[c]
                return qcarry

            lax.fori_loop(0, LANES, qbody, 0)
            return pltpu.async_copy(
                obuf, out_h.at[pl.ds(base + g * LANES, LANES)], semo)

        select(0, 0)
        fire(0, sem0)
        select(1, 1)
        fire(1, sem1)

        def pair(k, carry):
            g0 = k * 2
            g1 = g0 + 1
            drain(0, sem0)
            out0 = weighted_sum(g0, 0, semo0)
            select(g0 + 2, 0)
            fire(0, sem0)
            drain(1, sem1)
            out1 = weighted_sum(g1, 1, semo1)
            select(g1 + 2, 1)
            fire(1, sem1)
            out0.wait()
            out1.wait()
            return carry

        lax.fori_loop(0, groups // 2 - 1, pair, 0)
        drain(0, sem0)
        out0 = weighted_sum(groups - 2, 0, semo0)
        drain(1, sem1)
        out1 = weighted_sum(groups - 1, 1, semo1)
        out0.wait()
        out1.wait()

    return body(pack, xtt, zc2)


def kernel(xc, zc, xt, lengthscale_param):
    b, H, W, dx = xc.shape
    dz = zc.shape[-1]
    nt = xt.shape[1]
    N = H * W
    nq = b * nt
    lengthscale = 1e-05 + jax.nn.softplus(lengthscale_param)
    nhil = -0.5 / (lengthscale[0] * lengthscale[0])
    pack = jnp.concatenate([
        xc[0, :, 0, 0],
        xc[0, 0, :, 1],
        jnp.broadcast_to(nhil, (LANES,)).astype(jnp.float32),
    ])
    xtt = xt.reshape(nq, dx).T
    zc2 = zc.reshape(b * N, dz)
    out = _sc_decode(H, W, N, nq, nt, dz, pack, xtt, zc2)
    return out.reshape(b, nt, dz)
